# baseline (speedup 1.0000x reference)
import jax
import jax.numpy as jnp
from jax import lax
from jax.experimental import pallas as pl
from jax.experimental.pallas import tpu as pltpu
from jax.experimental.pallas import tpu_sc as plsc

N = 10000
D = 128
C = 40
CP = 48
NPAD = 10240
TRASH = N
NC = 2
NS = 16
NW = NC * NS
CHUNK = 128
NB = 4
NBP = 6
RPS = NPAD // NS
BLK = 1000


def _sc_degree(nch):
    def body(col_hbm, out_hbm, acc, idx_v, ones_v, zrow_v, dsem):
        cid = lax.axis_index("c")
        sid = lax.axis_index("s")
        wid = sid * NC + cid

        @pl.loop(0, CHUNK)
        def _(i):
            ones_v[i, pl.ds(0, 16)] = jnp.full((16,), 1.0, jnp.float32)
            zrow_v[i, pl.ds(0, 16)] = jnp.zeros((16,), jnp.float32)

        @pl.loop(0, RPS // CHUNK)
        def _(r):
            pltpu.sync_copy(zrow_v, acc.at[pl.ds(sid * RPS + r * CHUNK, CHUNK)])

        plsc.subcore_barrier()
        pltpu.sync_copy(col_hbm.at[wid], idx_v)

        for b in range(NB):
            pltpu.async_copy(ones_v, acc.at[idx_v.at[b]], dsem.at[b], add=True)

        @pl.loop(0, nch // NB - 1)
        def _(g):
            base = g * NB
            for b in range(NB):
                pltpu.make_async_copy(
                    ones_v, acc.at[idx_v.at[base + b]], dsem.at[b]).wait()
                pltpu.async_copy(ones_v, acc.at[idx_v.at[base + NB + b]],
                                 dsem.at[b], add=True)

        last = nch - NB
        for b in range(NB):
            pltpu.make_async_copy(
                ones_v, acc.at[idx_v.at[last + b]], dsem.at[b]).wait()

        plsc.subcore_barrier()
        pltpu.sync_copy(acc.at[pl.ds(sid * RPS, RPS)],
                        out_hbm.at[cid].at[pl.ds(sid * RPS, RPS)])

    return pl.kernel(
        body,
        out_type=jax.ShapeDtypeStruct((NC, NPAD, 16), jnp.float32),
        mesh=plsc.VectorSubcoreMesh(core_axis_name="c", subcore_axis_name="s",
                                    num_cores=NC, num_subcores=NS),
        scratch_types=[
            pltpu.VMEM_SHARED((NPAD, 16), jnp.float32),
            pltpu.VMEM((nch, CHUNK), jnp.int32),
            pltpu.VMEM((CHUNK, 16), jnp.float32),
            pltpu.VMEM((CHUNK, 16), jnp.float32),
            pltpu.SemaphoreType.DMA((NB,)),
        ],
        compiler_params=pltpu.CompilerParams(use_tc_tiling_on_sc=False),
    )


def _sc_propagate(nch):
    stage = N // NS
    stage0 = (stage // 8) * 8

    def body(hp_hbm, row_hbm, col_hbm, out_hbm,
             acc, hps, ridx_v, cidx_v, gbuf_v, zrow_v, gsem, ssem):
        cid = lax.axis_index("c")
        sid = lax.axis_index("s")
        wid = sid * NC + cid

        @pl.loop(0, CHUNK)
        def _(i):
            @pl.loop(0, CP, step=16)
            def _(cc):
                zrow_v[i, pl.ds(cc, 16)] = jnp.zeros((16,), jnp.float32)

        pltpu.sync_copy(hp_hbm.at[pl.ds(sid * stage0, stage0)],
                        hps.at[pl.ds(sid * stage0, stage0)])

        @pl.when(sid == NS - 1)
        def _():
            pltpu.sync_copy(hp_hbm.at[pl.ds(NS * stage0, N - NS * stage0)],
                            hps.at[pl.ds(NS * stage0, N - NS * stage0)])

        @pl.loop(0, RPS // CHUNK)
        def _(r):
            pltpu.sync_copy(zrow_v, acc.at[pl.ds(sid * RPS + r * CHUNK, CHUNK)])

        pltpu.sync_copy(row_hbm.at[wid], ridx_v)
        pltpu.sync_copy(col_hbm.at[wid], cidx_v)
        plsc.subcore_barrier()

        for b in range(NBP):
            pltpu.async_copy(hps.at[ridx_v.at[b]], gbuf_v.at[b], gsem.at[b])

        @pl.loop(0, nch // NBP - 1)
        def _(g):
            base = g * NBP
            for b in range(NBP):
                pltpu.make_async_copy(hps.at[ridx_v.at[base + b]],
                                      gbuf_v.at[b], gsem.at[b]).wait()
                pltpu.async_copy(gbuf_v.at[b], acc.at[cidx_v.at[base + b]],
                                 ssem.at[b], add=True)
            for b in range(NBP):
                pltpu.make_async_copy(gbuf_v.at[b], acc.at[cidx_v.at[base + b]],
                                      ssem.at[b]).wait()
                pltpu.async_copy(hps.at[ridx_v.at[base + NBP + b]],
                                 gbuf_v.at[b], gsem.at[b])

        last = nch - NBP
        for b in range(NBP):
            pltpu.make_async_copy(hps.at[ridx_v.at[last + b]],
                                  gbuf_v.at[b], gsem.at[b]).wait()
            pltpu.async_copy(gbuf_v.at[b], acc.at[cidx_v.at[last + b]],
                             ssem.at[b], add=True)
        for b in range(NBP):
            pltpu.make_async_copy(gbuf_v.at[b], acc.at[cidx_v.at[last + b]],
                                  ssem.at[b]).wait()

        plsc.subcore_barrier()
        pltpu.sync_copy(acc.at[pl.ds(sid * RPS, RPS)],
                        out_hbm.at[cid].at[pl.ds(sid * RPS, RPS)])

    return pl.kernel(
        body,
        out_type=jax.ShapeDtypeStruct((NC, NPAD, CP), jnp.float32),
        mesh=plsc.VectorSubcoreMesh(core_axis_name="c", subcore_axis_name="s",
                                    num_cores=NC, num_subcores=NS),
        scratch_types=[
            pltpu.VMEM_SHARED((NPAD, CP), jnp.float32),
            pltpu.VMEM_SHARED((N, CP), jnp.float32),
            pltpu.VMEM((nch, CHUNK), jnp.int32),
            pltpu.VMEM((nch, CHUNK), jnp.int32),
            pltpu.VMEM((NBP, CHUNK, CP), jnp.float32),
            pltpu.VMEM((CHUNK, CP), jnp.float32),
            pltpu.SemaphoreType.DMA((NBP,)),
            pltpu.SemaphoreType.DMA((NBP,)),
        ],
        compiler_params=pltpu.CompilerParams(use_tc_tiling_on_sc=False),
    )


def _tc_matmul(x, wt):
    def body(x_ref, w_ref, o_ref):
        o_ref[...] = jnp.dot(x_ref[...], w_ref[...],
                             preferred_element_type=jnp.float32)

    return pl.pallas_call(
        body,
        grid=(N // BLK,),
        in_specs=[pl.BlockSpec((BLK, D), lambda i: (i, 0)),
                  pl.BlockSpec((D, CP), lambda i: (0, 0))],
        out_specs=pl.BlockSpec((BLK, CP), lambda i: (i, 0)),
        out_shape=jax.ShapeDtypeStruct((N, CP), jnp.float32),
    )(x, wt)


def _tc_dis_scale(degs, y0):
    def body(da_ref, db_ref, y_ref, dis_ref, hp_ref):
        deg = da_ref[0, :, 0:1] + db_ref[0, :, 0:1] + 1.0
        dis = jnp.broadcast_to(lax.rsqrt(deg), (BLK, CP))
        dis_ref[...] = dis
        hp_ref[...] = y_ref[...] * dis

    return pl.pallas_call(
        body,
        grid=(N // BLK,),
        in_specs=[pl.BlockSpec((1, BLK, 16), lambda i: (0, i, 0)),
                  pl.BlockSpec((1, BLK, 16), lambda i: (1, i, 0)),
                  pl.BlockSpec((BLK, CP), lambda i: (i, 0))],
        out_specs=[pl.BlockSpec((BLK, CP), lambda i: (i, 0)),
                   pl.BlockSpec((BLK, CP), lambda i: (i, 0))],
        out_shape=[jax.ShapeDtypeStruct((N, CP), jnp.float32),
                   jax.ShapeDtypeStruct((N, CP), jnp.float32)],
    )(degs, degs, y0)


def _tc_mid_combine(p, hp, dis):
    def body(pa_ref, pb_ref, hp_ref, dis_ref, o_ref):
        d = dis_ref[...]
        o_ref[...] = d * d * (pa_ref[0] + pb_ref[0] + hp_ref[...])

    return pl.pallas_call(
        body,
        grid=(N // BLK,),
        in_specs=[pl.BlockSpec((1, BLK, CP), lambda i: (0, i, 0)),
                  pl.BlockSpec((1, BLK, CP), lambda i: (1, i, 0)),
                  pl.BlockSpec((BLK, CP), lambda i: (i, 0)),
                  pl.BlockSpec((BLK, CP), lambda i: (i, 0))],
        out_specs=pl.BlockSpec((BLK, CP), lambda i: (i, 0)),
        out_shape=jax.ShapeDtypeStruct((N, CP), jnp.float32),
    )(p, p, hp, dis)


def _tc_final(p, hp, dis, b2):
    def body(pa_ref, pb_ref, hp_ref, dis_ref, b_ref, o_ref):
        t = dis_ref[...] * (pa_ref[0] + pb_ref[0] + hp_ref[...])
        l = t[:, :C] + b_ref[...]
        m = jnp.max(l, axis=1, keepdims=True)
        e = jnp.exp(l - m)
        lse = jnp.log(jnp.sum(e, axis=1, keepdims=True)) + m
        o_ref[...] = l - lse

    return pl.pallas_call(
        body,
        grid=(N // BLK,),
        in_specs=[pl.BlockSpec((1, BLK, CP), lambda i: (0, i, 0)),
                  pl.BlockSpec((1, BLK, CP), lambda i: (1, i, 0)),
                  pl.BlockSpec((BLK, CP), lambda i: (i, 0)),
                  pl.BlockSpec((BLK, CP), lambda i: (i, 0)),
                  pl.BlockSpec((1, C), lambda i: (0, 0))],
        out_specs=pl.BlockSpec((BLK, C), lambda i: (i, 0)),
        out_shape=jax.ShapeDtypeStruct((N, C), jnp.float32),
    )(p, p, hp, dis, b2)


def kernel(x, edge_index, W, b):
    E = edge_index.shape[1]
    nch = -(-E // (NW * CHUNK))
    ring = NB * NBP // 2
    nch = -(-nch // ring) * ring
    ep = NW * nch * CHUNK
    row = edge_index[0].astype(jnp.int32)
    col = edge_index[1].astype(jnp.int32)
    npad_e = ep - E
    row_w = jnp.concatenate(
        [row, jnp.zeros((npad_e,), jnp.int32)]).reshape(NW, nch, CHUNK)
    trash_cols = TRASH + jnp.arange(npad_e, dtype=jnp.int32) % (NPAD - N)
    col_w = jnp.concatenate([col, trash_cols]).reshape(NW, nch, CHUNK)
    wt = jnp.zeros((D, CP), jnp.float32).at[:, :C].set(
        W.astype(jnp.float32).T)
    b2 = b.astype(jnp.float32).reshape(1, C)

    degs = _sc_degree(nch)(col_w)
    y0 = _tc_matmul(x.astype(jnp.float32), wt)
    dis48, h0p = _tc_dis_scale(degs, y0)

    prop = _sc_propagate(nch)
    p1 = prop(h0p, row_w, col_w)
    h1p = _tc_mid_combine(p1, h0p, dis48)
    p2 = prop(h1p, row_w, col_w)
    return _tc_final(p2, h1p, dis48, b2)


# speedup vs baseline: 1.0111x; 1.0111x over previous
import jax
import jax.numpy as jnp
from jax import lax
from jax.experimental import pallas as pl
from jax.experimental.pallas import tpu as pltpu
from jax.experimental.pallas import tpu_sc as plsc

N = 10000
D = 128
C = 40
CP = 48
NPAD = 10240
TRASH = N
NC = 2
NS = 16
NW = NC * NS
CHUNK = 128
NB = 4
NBP = 4
RPS = NPAD // NS
BLK = 1000


def _sc_degree(nch):
    def body(col_hbm, out_hbm, acc, idx_v, ones_v, zrow_v, dsem):
        cid = lax.axis_index("c")
        sid = lax.axis_index("s")
        wid = sid * NC + cid

        @pl.loop(0, CHUNK)
        def _(i):
            ones_v[i, pl.ds(0, 16)] = jnp.full((16,), 1.0, jnp.float32)
            zrow_v[i, pl.ds(0, 16)] = jnp.zeros((16,), jnp.float32)

        @pl.loop(0, RPS // CHUNK)
        def _(r):
            pltpu.sync_copy(zrow_v, acc.at[pl.ds(sid * RPS + r * CHUNK, CHUNK)])

        plsc.subcore_barrier()
        pltpu.sync_copy(col_hbm.at[wid], idx_v)

        for b in range(NB):
            pltpu.async_copy(ones_v, acc.at[idx_v.at[b]], dsem.at[b], add=True)

        @pl.loop(0, nch // NB - 1)
        def _(g):
            base = g * NB
            for b in range(NB):
                pltpu.make_async_copy(
                    ones_v, acc.at[idx_v.at[base + b]], dsem.at[b]).wait()
                pltpu.async_copy(ones_v, acc.at[idx_v.at[base + NB + b]],
                                 dsem.at[b], add=True)

        last = nch - NB
        for b in range(NB):
            pltpu.make_async_copy(
                ones_v, acc.at[idx_v.at[last + b]], dsem.at[b]).wait()

        plsc.subcore_barrier()
        pltpu.sync_copy(acc.at[pl.ds(sid * RPS, RPS)],
                        out_hbm.at[cid].at[pl.ds(sid * RPS, RPS)])

    return pl.kernel(
        body,
        out_type=jax.ShapeDtypeStruct((NC, NPAD, 16), jnp.float32),
        mesh=plsc.VectorSubcoreMesh(core_axis_name="c", subcore_axis_name="s",
                                    num_cores=NC, num_subcores=NS),
        scratch_types=[
            pltpu.VMEM_SHARED((NPAD, 16), jnp.float32),
            pltpu.VMEM((nch, CHUNK), jnp.int32),
            pltpu.VMEM((CHUNK, 16), jnp.float32),
            pltpu.VMEM((CHUNK, 16), jnp.float32),
            pltpu.SemaphoreType.DMA((NB,)),
        ],
        compiler_params=pltpu.CompilerParams(use_tc_tiling_on_sc=False),
    )


def _sc_propagate(nch):
    stage = N // NS
    stage0 = (stage // 8) * 8

    def body(hp_hbm, row_hbm, col_hbm, out_hbm,
             acc, hps, ridx_v, cidx_v, gbuf_v, zrow_v, gsem, ssem):
        cid = lax.axis_index("c")
        sid = lax.axis_index("s")
        wid = sid * NC + cid

        @pl.loop(0, CHUNK)
        def _(i):
            @pl.loop(0, CP, step=16)
            def _(cc):
                zrow_v[i, pl.ds(cc, 16)] = jnp.zeros((16,), jnp.float32)

        pltpu.sync_copy(hp_hbm.at[pl.ds(sid * stage0, stage0)],
                        hps.at[pl.ds(sid * stage0, stage0)])

        @pl.when(sid == NS - 1)
        def _():
            pltpu.sync_copy(hp_hbm.at[pl.ds(NS * stage0, N - NS * stage0)],
                            hps.at[pl.ds(NS * stage0, N - NS * stage0)])

        @pl.loop(0, RPS // CHUNK)
        def _(r):
            pltpu.sync_copy(zrow_v, acc.at[pl.ds(sid * RPS + r * CHUNK, CHUNK)])

        pltpu.sync_copy(row_hbm.at[wid], ridx_v)
        pltpu.sync_copy(col_hbm.at[wid], cidx_v)
        plsc.subcore_barrier()

        for b in range(NBP):
            pltpu.async_copy(hps.at[ridx_v.at[b]], gbuf_v.at[b], gsem.at[b])

        @pl.loop(0, nch // NBP - 1)
        def _(g):
            base = g * NBP
            for b in range(NBP):
                pltpu.make_async_copy(hps.at[ridx_v.at[base + b]],
                                      gbuf_v.at[b], gsem.at[b]).wait()
                pltpu.async_copy(gbuf_v.at[b], acc.at[cidx_v.at[base + b]],
                                 ssem.at[b], add=True)
            for b in range(NBP):
                pltpu.make_async_copy(gbuf_v.at[b], acc.at[cidx_v.at[base + b]],
                                      ssem.at[b]).wait()
                pltpu.async_copy(hps.at[ridx_v.at[base + NBP + b]],
                                 gbuf_v.at[b], gsem.at[b])

        last = nch - NBP
        for b in range(NBP):
            pltpu.make_async_copy(hps.at[ridx_v.at[last + b]],
                                  gbuf_v.at[b], gsem.at[b]).wait()
            pltpu.async_copy(gbuf_v.at[b], acc.at[cidx_v.at[last + b]],
                             ssem.at[b], add=True)
        for b in range(NBP):
            pltpu.make_async_copy(gbuf_v.at[b], acc.at[cidx_v.at[last + b]],
                                  ssem.at[b]).wait()

        plsc.subcore_barrier()
        pltpu.sync_copy(acc.at[pl.ds(sid * RPS, RPS)],
                        out_hbm.at[cid].at[pl.ds(sid * RPS, RPS)])

    return pl.kernel(
        body,
        out_type=jax.ShapeDtypeStruct((NC, NPAD, CP), jnp.float32),
        mesh=plsc.VectorSubcoreMesh(core_axis_name="c", subcore_axis_name="s",
                                    num_cores=NC, num_subcores=NS),
        scratch_types=[
            pltpu.VMEM_SHARED((NPAD, CP), jnp.float32),
            pltpu.VMEM_SHARED((N, CP), jnp.float32),
            pltpu.VMEM((nch, CHUNK), jnp.int32),
            pltpu.VMEM((nch, CHUNK), jnp.int32),
            pltpu.VMEM((NBP, CHUNK, CP), jnp.float32),
            pltpu.VMEM((CHUNK, CP), jnp.float32),
            pltpu.SemaphoreType.DMA((NBP,)),
            pltpu.SemaphoreType.DMA((NBP,)),
        ],
        compiler_params=pltpu.CompilerParams(use_tc_tiling_on_sc=False),
    )


def _tc_matmul(x, wt):
    def body(x_ref, w_ref, o_ref):
        o_ref[...] = jnp.dot(x_ref[...], w_ref[...],
                             preferred_element_type=jnp.float32)

    return pl.pallas_call(
        body,
        grid=(N // BLK,),
        in_specs=[pl.BlockSpec((BLK, D), lambda i: (i, 0)),
                  pl.BlockSpec((D, CP), lambda i: (0, 0))],
        out_specs=pl.BlockSpec((BLK, CP), lambda i: (i, 0)),
        out_shape=jax.ShapeDtypeStruct((N, CP), jnp.float32),
    )(x, wt)


def _tc_dis_scale(degs, y0):
    def body(da_ref, db_ref, y_ref, dis_ref, hp_ref):
        deg = da_ref[0, :, 0:1] + db_ref[0, :, 0:1] + 1.0
        dis = jnp.broadcast_to(lax.rsqrt(deg), (BLK, CP))
        dis_ref[...] = dis
        hp_ref[...] = y_ref[...] * dis

    return pl.pallas_call(
        body,
        grid=(N // BLK,),
        in_specs=[pl.BlockSpec((1, BLK, 16), lambda i: (0, i, 0)),
                  pl.BlockSpec((1, BLK, 16), lambda i: (1, i, 0)),
                  pl.BlockSpec((BLK, CP), lambda i: (i, 0))],
        out_specs=[pl.BlockSpec((BLK, CP), lambda i: (i, 0)),
                   pl.BlockSpec((BLK, CP), lambda i: (i, 0))],
        out_shape=[jax.ShapeDtypeStruct((N, CP), jnp.float32),
                   jax.ShapeDtypeStruct((N, CP), jnp.float32)],
    )(degs, degs, y0)


def _tc_mid_combine(p, hp, dis):
    def body(pa_ref, pb_ref, hp_ref, dis_ref, o_ref):
        d = dis_ref[...]
        o_ref[...] = d * d * (pa_ref[0] + pb_ref[0] + hp_ref[...])

    return pl.pallas_call(
        body,
        grid=(N // BLK,),
        in_specs=[pl.BlockSpec((1, BLK, CP), lambda i: (0, i, 0)),
                  pl.BlockSpec((1, BLK, CP), lambda i: (1, i, 0)),
                  pl.BlockSpec((BLK, CP), lambda i: (i, 0)),
                  pl.BlockSpec((BLK, CP), lambda i: (i, 0))],
        out_specs=pl.BlockSpec((BLK, CP), lambda i: (i, 0)),
        out_shape=jax.ShapeDtypeStruct((N, CP), jnp.float32),
    )(p, p, hp, dis)


def _tc_final(p, hp, dis, b2):
    def body(pa_ref, pb_ref, hp_ref, dis_ref, b_ref, o_ref):
        t = dis_ref[...] * (pa_ref[0] + pb_ref[0] + hp_ref[...])
        l = t[:, :C] + b_ref[...]
        m = jnp.max(l, axis=1, keepdims=True)
        e = jnp.exp(l - m)
        lse = jnp.log(jnp.sum(e, axis=1, keepdims=True)) + m
        o_ref[...] = l - lse

    return pl.pallas_call(
        body,
        grid=(N // BLK,),
        in_specs=[pl.BlockSpec((1, BLK, CP), lambda i: (0, i, 0)),
                  pl.BlockSpec((1, BLK, CP), lambda i: (1, i, 0)),
                  pl.BlockSpec((BLK, CP), lambda i: (i, 0)),
                  pl.BlockSpec((BLK, CP), lambda i: (i, 0)),
                  pl.BlockSpec((1, C), lambda i: (0, 0))],
        out_specs=pl.BlockSpec((BLK, C), lambda i: (i, 0)),
        out_shape=jax.ShapeDtypeStruct((N, C), jnp.float32),
    )(p, p, hp, dis, b2)


def kernel(x, edge_index, W, b):
    E = edge_index.shape[1]
    nch = -(-E // (NW * CHUNK))
    ring = NB * NBP // 2
    nch = -(-nch // ring) * ring
    ep = NW * nch * CHUNK
    row = edge_index[0].astype(jnp.int32)
    col = edge_index[1].astype(jnp.int32)
    npad_e = ep - E
    row_w = jnp.concatenate(
        [row, jnp.zeros((npad_e,), jnp.int32)]).reshape(NW, nch, CHUNK)
    trash_cols = TRASH + jnp.arange(npad_e, dtype=jnp.int32) % (NPAD - N)
    col_w = jnp.concatenate([col, trash_cols]).reshape(NW, nch, CHUNK)
    wt = jnp.zeros((D, CP), jnp.float32).at[:, :C].set(
        W.astype(jnp.float32).T)
    b2 = b.astype(jnp.float32).reshape(1, C)

    degs = _sc_degree(nch)(col_w)
    y0 = _tc_matmul(x.astype(jnp.float32), wt)
    dis48, h0p = _tc_dis_scale(degs, y0)

    prop = _sc_propagate(nch)
    p1 = prop(h0p, row_w, col_w)
    h1p = _tc_mid_combine(p1, h0p, dis48)
    p2 = prop(h1p, row_w, col_w)
    return _tc_final(p2, h1p, dis48, b2)


# speedup vs baseline: 1.0136x; 1.0025x over previous
import jax
import jax.numpy as jnp
from jax import lax
from jax.experimental import pallas as pl
from jax.experimental.pallas import tpu as pltpu
from jax.experimental.pallas import tpu_sc as plsc

N = 10000
D = 128
C = 40
CP = 48
NPAD = 10240
TRASH = N
NC = 2
NS = 16
NW = NC * NS
CHUNK = 128
NB = 4
NBP = 4
RPS = NPAD // NS
BLK = 1000


def _sc_degree(nch):
    def body(col_hbm, out_hbm, acc, idx_v, ones_v, zrow_v, dsem):
        cid = lax.axis_index("c")
        sid = lax.axis_index("s")
        wid = sid * NC + cid

        @pl.loop(0, CHUNK)
        def _(i):
            ones_v[i, pl.ds(0, 16)] = jnp.full((16,), 1.0, jnp.float32)
            zrow_v[i, pl.ds(0, 16)] = jnp.zeros((16,), jnp.float32)

        @pl.loop(0, RPS // CHUNK)
        def _(r):
            pltpu.sync_copy(zrow_v, acc.at[pl.ds(sid * RPS + r * CHUNK, CHUNK)])

        plsc.subcore_barrier()
        pltpu.sync_copy(col_hbm.at[wid], idx_v)

        for b in range(NB):
            pltpu.async_copy(ones_v, acc.at[idx_v.at[b]], dsem.at[b], add=True)

        @pl.loop(0, nch // NB - 1)
        def _(g):
            base = g * NB
            for b in range(NB):
                pltpu.make_async_copy(
                    ones_v, acc.at[idx_v.at[base + b]], dsem.at[b]).wait()
                pltpu.async_copy(ones_v, acc.at[idx_v.at[base + NB + b]],
                                 dsem.at[b], add=True)

        last = nch - NB
        for b in range(NB):
            pltpu.make_async_copy(
                ones_v, acc.at[idx_v.at[last + b]], dsem.at[b]).wait()

        plsc.subcore_barrier()
        pltpu.sync_copy(acc.at[pl.ds(sid * RPS, RPS)],
                        out_hbm.at[cid].at[pl.ds(sid * RPS, RPS)])

    return pl.kernel(
        body,
        out_type=jax.ShapeDtypeStruct((NC, NPAD, 16), jnp.float32),
        mesh=plsc.VectorSubcoreMesh(core_axis_name="c", subcore_axis_name="s",
                                    num_cores=NC, num_subcores=NS),
        scratch_types=[
            pltpu.VMEM_SHARED((NPAD, 16), jnp.float32),
            pltpu.VMEM((nch, CHUNK), jnp.int32),
            pltpu.VMEM((CHUNK, 16), jnp.float32),
            pltpu.VMEM((CHUNK, 16), jnp.float32),
            pltpu.SemaphoreType.DMA((NB,)),
        ],
        compiler_params=pltpu.CompilerParams(use_tc_tiling_on_sc=False),
    )


def _sc_propagate(nch):
    stage = N // NS
    stage0 = (stage // 8) * 8

    def body(hp_hbm, row_hbm, col_hbm, out_hbm,
             acc, hps, ridx_v, cidx_v, gbuf_v, zrow_v, gsem, ssem):
        cid = lax.axis_index("c")
        sid = lax.axis_index("s")
        wid = sid * NC + cid

        @pl.loop(0, CHUNK)
        def _(i):
            @pl.loop(0, CP, step=16)
            def _(cc):
                zrow_v[i, pl.ds(cc, 16)] = jnp.zeros((16,), jnp.float32)

        pltpu.sync_copy(hp_hbm.at[pl.ds(sid * stage0, stage0)],
                        hps.at[pl.ds(sid * stage0, stage0)])

        @pl.when(sid == NS - 1)
        def _():
            pltpu.sync_copy(hp_hbm.at[pl.ds(NS * stage0, N - NS * stage0)],
                            hps.at[pl.ds(NS * stage0, N - NS * stage0)])

        @pl.loop(0, RPS // CHUNK)
        def _(r):
            pltpu.sync_copy(zrow_v, acc.at[pl.ds(sid * RPS + r * CHUNK, CHUNK)])

        pltpu.sync_copy(row_hbm.at[wid], ridx_v)
        pltpu.sync_copy(col_hbm.at[wid], cidx_v)
        plsc.subcore_barrier()

        for b in range(NBP):
            pltpu.async_copy(hps.at[ridx_v.at[b]], gbuf_v.at[b], gsem.at[b])

        @pl.loop(0, nch // NBP - 1)
        def _(g):
            base = g * NBP
            for b in range(NBP):
                pltpu.make_async_copy(hps.at[ridx_v.at[base + b]],
                                      gbuf_v.at[b], gsem.at[b]).wait()
                pltpu.async_copy(gbuf_v.at[b], acc.at[cidx_v.at[base + b]],
                                 ssem.at[b], add=True)
            for b in range(NBP):
                pltpu.make_async_copy(gbuf_v.at[b], acc.at[cidx_v.at[base + b]],
                                      ssem.at[b]).wait()
                pltpu.async_copy(hps.at[ridx_v.at[base + NBP + b]],
                                 gbuf_v.at[b], gsem.at[b])

        last = nch - NBP
        for b in range(NBP):
            pltpu.make_async_copy(hps.at[ridx_v.at[last + b]],
                                  gbuf_v.at[b], gsem.at[b]).wait()
            pltpu.async_copy(gbuf_v.at[b], acc.at[cidx_v.at[last + b]],
                             ssem.at[b], add=True)
        for b in range(NBP):
            pltpu.make_async_copy(gbuf_v.at[b], acc.at[cidx_v.at[last + b]],
                                  ssem.at[b]).wait()

        plsc.subcore_barrier()
        pltpu.sync_copy(acc.at[pl.ds(sid * RPS, RPS)],
                        out_hbm.at[cid].at[pl.ds(sid * RPS, RPS)])

    return pl.kernel(
        body,
        out_type=jax.ShapeDtypeStruct((NC, NPAD, CP), jnp.float32),
        mesh=plsc.VectorSubcoreMesh(core_axis_name="c", subcore_axis_name="s",
                                    num_cores=NC, num_subcores=NS),
        scratch_types=[
            pltpu.VMEM_SHARED((NPAD, CP), jnp.float32),
            pltpu.VMEM_SHARED((N, CP), jnp.float32),
            pltpu.VMEM((nch, CHUNK), jnp.int32),
            pltpu.VMEM((nch, CHUNK), jnp.int32),
            pltpu.VMEM((NBP, CHUNK, CP), jnp.float32),
            pltpu.VMEM((CHUNK, CP), jnp.float32),
            pltpu.SemaphoreType.DMA((NBP,)),
            pltpu.SemaphoreType.DMA((NBP,)),
        ],
        compiler_params=pltpu.CompilerParams(use_tc_tiling_on_sc=False),
    )


def _tc_matmul(x, wp):
    def body(x_ref, w_ref, o_ref):
        o_ref[...] = lax.dot_general(
            x_ref[...], w_ref[...], (((1,), (1,)), ((), ())),
            preferred_element_type=jnp.float32)

    return pl.pallas_call(
        body,
        grid=(N // BLK,),
        in_specs=[pl.BlockSpec((BLK, D), lambda i: (i, 0)),
                  pl.BlockSpec((CP, D), lambda i: (0, 0))],
        out_specs=pl.BlockSpec((BLK, CP), lambda i: (i, 0)),
        out_shape=jax.ShapeDtypeStruct((N, CP), jnp.float32),
    )(x, wp)


def _tc_dis_scale(degs, y0):
    def body(da_ref, db_ref, y_ref, dis_ref, hp_ref):
        deg = da_ref[0, :, 0:1] + db_ref[0, :, 0:1] + 1.0
        dis = jnp.broadcast_to(lax.rsqrt(deg), (BLK, CP))
        dis_ref[...] = dis
        hp_ref[...] = y_ref[...] * dis

    return pl.pallas_call(
        body,
        grid=(N // BLK,),
        in_specs=[pl.BlockSpec((1, BLK, 16), lambda i: (0, i, 0)),
                  pl.BlockSpec((1, BLK, 16), lambda i: (1, i, 0)),
                  pl.BlockSpec((BLK, CP), lambda i: (i, 0))],
        out_specs=[pl.BlockSpec((BLK, CP), lambda i: (i, 0)),
                   pl.BlockSpec((BLK, CP), lambda i: (i, 0))],
        out_shape=[jax.ShapeDtypeStruct((N, CP), jnp.float32),
                   jax.ShapeDtypeStruct((N, CP), jnp.float32)],
    )(degs, degs, y0)


def _tc_mid_combine(p, hp, dis):
    def body(pa_ref, pb_ref, hp_ref, dis_ref, o_ref):
        d = dis_ref[...]
        o_ref[...] = d * d * (pa_ref[0] + pb_ref[0] + hp_ref[...])

    return pl.pallas_call(
        body,
        grid=(N // BLK,),
        in_specs=[pl.BlockSpec((1, BLK, CP), lambda i: (0, i, 0)),
                  pl.BlockSpec((1, BLK, CP), lambda i: (1, i, 0)),
                  pl.BlockSpec((BLK, CP), lambda i: (i, 0)),
                  pl.BlockSpec((BLK, CP), lambda i: (i, 0))],
        out_specs=pl.BlockSpec((BLK, CP), lambda i: (i, 0)),
        out_shape=jax.ShapeDtypeStruct((N, CP), jnp.float32),
    )(p, p, hp, dis)


def _tc_final(p, hp, dis, b2):
    def body(pa_ref, pb_ref, hp_ref, dis_ref, b_ref, o_ref):
        t = dis_ref[...] * (pa_ref[0] + pb_ref[0] + hp_ref[...])
        l = t[:, :C] + b_ref[...]
        m = jnp.max(l, axis=1, keepdims=True)
        e = jnp.exp(l - m)
        lse = jnp.log(jnp.sum(e, axis=1, keepdims=True)) + m
        o_ref[...] = l - lse

    return pl.pallas_call(
        body,
        grid=(N // BLK,),
        in_specs=[pl.BlockSpec((1, BLK, CP), lambda i: (0, i, 0)),
                  pl.BlockSpec((1, BLK, CP), lambda i: (1, i, 0)),
                  pl.BlockSpec((BLK, CP), lambda i: (i, 0)),
                  pl.BlockSpec((BLK, CP), lambda i: (i, 0)),
                  pl.BlockSpec((1, C), lambda i: (0, 0))],
        out_specs=pl.BlockSpec((BLK, C), lambda i: (i, 0)),
        out_shape=jax.ShapeDtypeStruct((N, C), jnp.float32),
    )(p, p, hp, dis, b2)


def kernel(x, edge_index, W, b):
    E = edge_index.shape[1]
    nch = -(-E // (NW * CHUNK))
    ring = NB * NBP // 2
    nch = -(-nch // ring) * ring
    ep = NW * nch * CHUNK
    row = edge_index[0].astype(jnp.int32)
    col = edge_index[1].astype(jnp.int32)
    npad_e = ep - E
    row_w = jnp.concatenate(
        [row, jnp.zeros((npad_e,), jnp.int32)]).reshape(NW, nch, CHUNK)
    trash_cols = TRASH + jnp.arange(npad_e, dtype=jnp.int32) % (NPAD - N)
    col_w = jnp.concatenate([col, trash_cols]).reshape(NW, nch, CHUNK)
    wp = jnp.zeros((CP, D), jnp.float32).at[:C].set(W.astype(jnp.float32))
    b2 = b.astype(jnp.float32).reshape(1, C)

    degs = _sc_degree(nch)(col_w)
    y0 = _tc_matmul(x.astype(jnp.float32), wp)
    dis48, h0p = _tc_dis_scale(degs, y0)

    prop = _sc_propagate(nch)
    p1 = prop(h0p, row_w, col_w)
    h1p = _tc_mid_combine(p1, h0p, dis48)
    p2 = prop(h1p, row_w, col_w)
    return _tc_final(p2, h1p, dis48, b2)
